# baseline (speedup 1.0000x reference)
import jax
import jax.numpy as jnp
from jax.experimental import pallas as pl
from jax.experimental.pallas import tpu as pltpu

_B, _H, _W, _C = 16, 32, 32, 768
_ROWS = _H * _W
_BR = 128


def _body(params_ref, x_ref, scale_ref, mean_ref, out_ref):
    l2t0 = params_ref[0]
    l2r = params_ref[1]
    inv_l2r = params_ref[2]
    l2b0 = params_ref[3]
    s = jnp.abs(scale_ref[...])
    idx = jnp.ceil((jnp.log2(s) - l2b0) * inv_l2r)
    idx = jnp.clip(idx, 0.0, 63.0)
    q = jnp.exp2(l2t0 + idx * l2r)
    m = mean_ref[...]
    x = x_ref[...]
    qb = q[None, :, :]
    mb = m[None, :, :]
    out_ref[...] = jnp.round((x - mb) / qb) * qb + mb


def kernel(inputs, scale, mean, scale_table, midpoints):
    x = inputs.reshape(_B, _ROWS, _C)
    s = scale.reshape(_ROWS, _C)
    m = mean.reshape(_ROWS, _C)

    n = scale_table.shape[0]
    l2t0 = jnp.log2(scale_table[0])
    l2r = (jnp.log2(scale_table[n - 1]) - l2t0) / (n - 1)
    l2b0 = jnp.log2(midpoints[0])
    params = jnp.stack([l2t0, l2r, 1.0 / l2r, l2b0]).astype(jnp.float32)

    grid = (_ROWS // _BR,)
    out = pl.pallas_call(
        _body,
        grid=grid,
        in_specs=[
            pl.BlockSpec(memory_space=pltpu.SMEM),
            pl.BlockSpec((_B, _BR, _C), lambda i: (0, i, 0)),
            pl.BlockSpec((_BR, _C), lambda i: (i, 0)),
            pl.BlockSpec((_BR, _C), lambda i: (i, 0)),
        ],
        out_specs=pl.BlockSpec((_B, _BR, _C), lambda i: (0, i, 0)),
        out_shape=jax.ShapeDtypeStruct((_B, _ROWS, _C), jnp.float32),
        compiler_params=pltpu.CompilerParams(
            dimension_semantics=("arbitrary",),
        ),
    )(params, x, s, m)
    return out.reshape(_B, _H, _W, _C)


# speedup vs baseline: 1.0135x; 1.0135x over previous
import jax
import jax.numpy as jnp
from jax.experimental import pallas as pl
from jax.experimental.pallas import tpu as pltpu

_B, _H, _W, _C = 16, 32, 32, 768
_ROWS = _H * _W
_BR = 128


def _body(table_ref, mid_ref, x_ref, scale_ref, mean_ref, out_ref):
    s = jnp.abs(scale_ref[...])
    q = jnp.full(s.shape, table_ref[0], dtype=jnp.float32)
    for j in range(mid_ref.shape[0]):
        q = jnp.where(s > mid_ref[j], table_ref[j + 1], q)
    m = mean_ref[...]
    x = x_ref[...]
    qb = q[None, :, :]
    mb = m[None, :, :]
    out_ref[...] = jnp.round((x - mb) / qb) * qb + mb


def kernel(inputs, scale, mean, scale_table, midpoints):
    x = inputs.reshape(_B, _ROWS, _C)
    s = scale.reshape(_ROWS, _C)
    m = mean.reshape(_ROWS, _C)

    grid = (_ROWS // _BR,)
    out = pl.pallas_call(
        _body,
        grid=grid,
        in_specs=[
            pl.BlockSpec(memory_space=pltpu.SMEM),
            pl.BlockSpec(memory_space=pltpu.SMEM),
            pl.BlockSpec((_B, _BR, _C), lambda i: (0, i, 0)),
            pl.BlockSpec((_BR, _C), lambda i: (i, 0)),
            pl.BlockSpec((_BR, _C), lambda i: (i, 0)),
        ],
        out_specs=pl.BlockSpec((_B, _BR, _C), lambda i: (0, i, 0)),
        out_shape=jax.ShapeDtypeStruct((_B, _ROWS, _C), jnp.float32),
        compiler_params=pltpu.CompilerParams(
            dimension_semantics=("arbitrary",),
        ),
    )(scale_table, midpoints, x, s, m)
    return out.reshape(_B, _H, _W, _C)
